# baseline (speedup 1.0000x reference)
import functools

import jax
import jax.numpy as jnp
from jax import lax
from jax.experimental import pallas as pl
from jax.experimental.pallas import tpu as pltpu
from jax.experimental.pallas import tpu_sc as plsc

PAD_IDX = 1
LN_EPS = 1e-05
LANES = 16


def _posid_body(ids_ref, out_ref):
    ids = ids_ref[...]
    mask = (ids != PAD_IDX).astype(jnp.int32)
    s = ids.shape[1]
    inc = mask
    d = 1
    while d < s:
        shifted = jnp.concatenate(
            [jnp.zeros((ids.shape[0], d), jnp.int32), inc[:, :-d]], axis=1)
        inc = inc + shifted
        d *= 2
    out_ref[...] = inc * mask + PAD_IDX


def _make_sc_kernel(n_tokens, hidden, chunk, n_workers):
    tokens_per_worker = n_tokens // n_workers
    n_chunks = tokens_per_worker // chunk
    n_slices = hidden // LANES
    mesh = plsc.VectorSubcoreMesh(core_axis_name="c", subcore_axis_name="s")

    @functools.partial(
        pl.kernel,
        mesh=mesh,
        out_type=jax.ShapeDtypeStruct((n_tokens, hidden), jnp.float32),
        scratch_types=[
            pltpu.VMEM((chunk,), jnp.int32),
            pltpu.VMEM((chunk,), jnp.int32),
            pltpu.VMEM((chunk,), jnp.int32),
            pltpu.VMEM((chunk, hidden), jnp.float32),
            pltpu.VMEM((chunk, hidden), jnp.float32),
            pltpu.VMEM((chunk, hidden), jnp.float32),
            pltpu.VMEM((hidden,), jnp.float32),
            pltpu.VMEM((hidden,), jnp.float32),
            pltpu.VMEM((hidden,), jnp.float32),
            pltpu.VMEM((2 * LANES,), jnp.float32),
            pltpu.SemaphoreType.DMA,
        ],
    )
    def sc_kernel(word_ids_hbm, spk_ids_hbm, pos_ids_hbm,
                  word_hbm, pos_hbm, type_hbm, spk_hbm,
                  gamma_hbm, beta_hbm, out_hbm,
                  idx_w, idx_s, idx_p, buf_w, buf_s, buf_p,
                  g_v, b_v, t_v, red_v, sem):
        wid = lax.axis_index("s") * 2 + lax.axis_index("c")
        base = wid * tokens_per_worker

        pltpu.sync_copy(gamma_hbm, g_v)
        pltpu.sync_copy(beta_hbm, b_v)
        pltpu.sync_copy(type_hbm.at[0], t_v)

        def chunk_body(ci, carry):
            off = base + ci * chunk
            pltpu.sync_copy(word_ids_hbm.at[pl.ds(off, chunk)], idx_w)
            pltpu.sync_copy(spk_ids_hbm.at[pl.ds(off, chunk)], idx_s)
            pltpu.sync_copy(pos_ids_hbm.at[pl.ds(off, chunk)], idx_p)
            cp_w = pltpu.async_copy(word_hbm.at[idx_w], buf_w, sem)
            cp_s = pltpu.async_copy(spk_hbm.at[idx_s], buf_s, sem)
            cp_p = pltpu.async_copy(pos_hbm.at[idx_p], buf_p, sem)
            cp_w.wait()
            cp_s.wait()
            cp_p.wait()

            def tok_body(t, tc):
                acc_s = jnp.zeros((LANES,), jnp.float32)
                acc_q = jnp.zeros((LANES,), jnp.float32)
                for j in range(n_slices):
                    sl = pl.ds(j * LANES, LANES)
                    x = buf_w[t, sl] + buf_s[t, sl] + buf_p[t, sl] + t_v[sl]
                    buf_w[t, sl] = x
                    acc_s = acc_s + x
                    acc_q = acc_q + x * x
                ssum = acc_s[0]
                qsum = acc_q[0]
                for i in range(1, LANES):
                    ssum = ssum + acc_s[i]
                    qsum = qsum + acc_q[i]
                mean = ssum * (1.0 / hidden)
                var = qsum * (1.0 / hidden) - mean * mean
                v = var + LN_EPS
                vi = lax.bitcast_convert_type(v, jnp.int32)
                y = lax.bitcast_convert_type(
                    jnp.int32(0x5F3759DF) - (vi >> 1), jnp.float32)
                hv = 0.5 * v
                y = y * (1.5 - hv * y * y)
                y = y * (1.5 - hv * y * y)
                y = y * (1.5 - hv * y * y)
                for j in range(n_slices):
                    sl = pl.ds(j * LANES, LANES)
                    x = buf_w[t, sl]
                    buf_w[t, sl] = (x - mean) * y * g_v[sl] + b_v[sl]
                return tc

            lax.fori_loop(0, chunk, tok_body, 0)
            pltpu.sync_copy(buf_w, out_hbm.at[pl.ds(off, chunk)])
            return carry

        lax.fori_loop(0, n_chunks, chunk_body, 0)

    return sc_kernel


def kernel(input_ids, speaker_ids, word_table, pos_table, type_table,
           speaker_table, ln_gamma, ln_beta):
    b, s = input_ids.shape
    hidden = word_table.shape[1]
    n = b * s

    pos_ids = pl.pallas_call(
        _posid_body,
        out_shape=jax.ShapeDtypeStruct((b, s), jnp.int32),
    )(input_ids.astype(jnp.int32))

    sc = _make_sc_kernel(n, hidden, chunk=32, n_workers=32)
    out = sc(
        input_ids.astype(jnp.int32).reshape(n),
        speaker_ids.astype(jnp.int32).reshape(n),
        pos_ids.reshape(n),
        word_table, pos_table, type_table, speaker_table,
        ln_gamma, ln_beta,
    )
    return out.reshape(b, s, hidden)


# speedup vs baseline: 2.2144x; 2.2144x over previous
import functools

import jax
import jax.numpy as jnp
from jax import lax
from jax.experimental import pallas as pl
from jax.experimental.pallas import tpu as pltpu
from jax.experimental.pallas import tpu_sc as plsc

PAD_IDX = 1
LN_EPS = 1e-05
LANES = 16
CHUNK = 16


def _prep_body(ids_ref, pos_ref, type_ref, posid_ref, posplus_ref):
    ids = ids_ref[...]
    mask = (ids != PAD_IDX).astype(jnp.int32)
    s = ids.shape[1]
    inc = mask
    d = 1
    while d < s:
        shifted = jnp.concatenate(
            [jnp.zeros((ids.shape[0], d), jnp.int32), inc[:, :-d]], axis=1)
        inc = inc + shifted
        d *= 2
    posid_ref[...] = inc * mask + PAD_IDX
    posplus_ref[...] = pos_ref[...] + type_ref[0, :][None, :]


def _make_sc_kernel(n_tokens, hidden, n_workers):
    tpw = n_tokens // n_workers
    n_chunks = tpw // CHUNK
    n_slices = hidden // LANES
    mesh = plsc.VectorSubcoreMesh(core_axis_name="c", subcore_axis_name="s")

    @functools.partial(
        pl.kernel,
        mesh=mesh,
        out_type=jax.ShapeDtypeStruct((n_tokens, hidden), jnp.float32),
        scratch_types=[
            pltpu.VMEM((tpw,), jnp.int32),
            pltpu.VMEM((tpw,), jnp.int32),
            pltpu.VMEM((tpw,), jnp.int32),
            pltpu.VMEM((CHUNK, hidden), jnp.float32),
            pltpu.VMEM((CHUNK, hidden), jnp.float32),
            pltpu.VMEM((CHUNK, hidden), jnp.float32),
            pltpu.VMEM((CHUNK, hidden), jnp.float32),
            pltpu.VMEM((CHUNK, hidden), jnp.float32),
            pltpu.VMEM((CHUNK, hidden), jnp.float32),
            pltpu.VMEM((CHUNK, hidden), jnp.float32),
            pltpu.VMEM((CHUNK, hidden), jnp.float32),
            pltpu.VMEM((hidden,), jnp.float32),
            pltpu.VMEM((hidden,), jnp.float32),
            pltpu.SMEM((2 * LANES,), jnp.float32),
            pltpu.SemaphoreType.DMA,
            pltpu.SemaphoreType.DMA,
            pltpu.SemaphoreType.DMA,
            pltpu.SemaphoreType.DMA,
        ],
    )
    def sc_kernel(word_ids_hbm, spk_ids_hbm, pos_ids_hbm,
                  word_hbm, pos_hbm, spk_hbm, gamma_hbm, beta_hbm,
                  out_hbm,
                  vw, vs, vp, bw0, bw1, bs0, bs1, bp0, bp1, o0, o1,
                  g_v, b_v, stats, sem_g0, sem_g1, sem_o0, sem_o1):
        wid = lax.axis_index("s") * 2 + lax.axis_index("c")
        base = wid * tpw
        bufs = ((bw0, bs0, bp0, o0, sem_g0, sem_o0),
                (bw1, bs1, bp1, o1, sem_g1, sem_o1))

        pltpu.sync_copy(word_ids_hbm.at[pl.ds(base, tpw)], vw)
        pltpu.sync_copy(spk_ids_hbm.at[pl.ds(base, tpw)], vs)
        pltpu.sync_copy(pos_ids_hbm.at[pl.ds(base, tpw)], vp)
        pltpu.sync_copy(gamma_hbm, g_v)
        pltpu.sync_copy(beta_hbm, b_v)

        def issue_gathers(c, parity):
            bw, bs, bp, _, sem, _ = bufs[parity]
            off = c * CHUNK
            pltpu.async_copy(word_hbm.at[vw[pl.ds(off, CHUNK)]], bw, sem)
            pltpu.async_copy(spk_hbm.at[vs[pl.ds(off, CHUNK)]], bs, sem)
            pltpu.async_copy(pos_hbm.at[vp[pl.ds(off, CHUNK)]], bp, sem)

        def wait_gathers(parity):
            bw, bs, bp, _, sem, _ = bufs[parity]
            pltpu.make_async_copy(word_hbm.at[vw[pl.ds(0, CHUNK)]], bw, sem).wait()
            pltpu.make_async_copy(word_hbm.at[vw[pl.ds(0, CHUNK)]], bs, sem).wait()
            pltpu.make_async_copy(word_hbm.at[vw[pl.ds(0, CHUNK)]], bp, sem).wait()

        def wait_out(parity):
            _, _, _, o, _, sem = bufs[parity]
            pltpu.make_async_copy(o, out_hbm.at[pl.ds(base, CHUNK)], sem).wait()

        def compute_chunk(c, parity):
            bw, bs, bp, o, _, sem_o = bufs[parity]
            off = base + c * CHUNK

            def tok_body(t, carry):
                acc_s = jnp.zeros((LANES,), jnp.float32)
                acc_q = jnp.zeros((LANES,), jnp.float32)
                for j in range(n_slices):
                    sl = pl.ds(j * LANES, LANES)
                    x = bw[t, sl] + bs[t, sl] + bp[t, sl]
                    o[t, sl] = x
                    acc_s = acc_s + x
                    acc_q = acc_q + x * x
                ssum = acc_s[0]
                qsum = acc_q[0]
                for i in range(1, LANES):
                    ssum = ssum + acc_s[i]
                    qsum = qsum + acc_q[i]
                mean = ssum * (1.0 / hidden)
                var = qsum * (1.0 / hidden) - mean * mean
                v = var + LN_EPS
                vi = lax.bitcast_convert_type(v, jnp.int32)
                y = lax.bitcast_convert_type(
                    jnp.int32(0x5F3759DF) - (vi >> 1), jnp.float32)
                hv = 0.5 * v
                y = y * (1.5 - hv * y * y)
                y = y * (1.5 - hv * y * y)
                y = y * (1.5 - hv * y * y)
                stats[t] = mean
                stats[t + LANES] = y
                return carry

            lax.fori_loop(0, CHUNK, tok_body, 0)

            def slice_body(j, carry):
                sl = pl.ds(j * LANES, LANES)
                g = g_v[sl]
                b = b_v[sl]
                for t in range(CHUNK):
                    x = o[t, sl]
                    o[t, sl] = (x - stats[t]) * stats[t + LANES] * g + b
                return carry

            lax.fori_loop(0, n_slices, slice_body, 0)
            pltpu.async_copy(o, out_hbm.at[pl.ds(off, CHUNK)], sem_o)

        issue_gathers(0, 0)

        def pair_body(k, carry):
            cA = 2 * k
            wait_gathers(0)
            issue_gathers(cA + 1, 1)

            @pl.when(k > 0)
            def _():
                wait_out(0)

            compute_chunk(cA, 0)

            wait_gathers(1)

            @pl.when(k < n_chunks // 2 - 1)
            def _():
                issue_gathers(cA + 2, 0)

            @pl.when(k > 0)
            def _():
                wait_out(1)

            compute_chunk(cA + 1, 1)
            return carry

        lax.fori_loop(0, n_chunks // 2, pair_body, 0)
        wait_out(0)
        wait_out(1)

    return sc_kernel


def kernel(input_ids, speaker_ids, word_table, pos_table, type_table,
           speaker_table, ln_gamma, ln_beta):
    b, s = input_ids.shape
    hidden = word_table.shape[1]
    max_pos = pos_table.shape[0]
    n = b * s

    pos_ids, pos_plus = pl.pallas_call(
        _prep_body,
        out_shape=(
            jax.ShapeDtypeStruct((b, s), jnp.int32),
            jax.ShapeDtypeStruct((max_pos, hidden), jnp.float32),
        ),
    )(input_ids.astype(jnp.int32), pos_table, type_table)

    sc = _make_sc_kernel(n, hidden, n_workers=32)
    out = sc(
        input_ids.astype(jnp.int32).reshape(n),
        speaker_ids.astype(jnp.int32).reshape(n),
        pos_ids.reshape(n),
        word_table, pos_plus, speaker_table,
        ln_gamma, ln_beta,
    )
    return out.reshape(b, s, hidden)
